# baseline (speedup 1.0000x reference)
import jax
import jax.numpy as jnp
from jax.experimental import pallas as pl
from jax.experimental.pallas import tpu as pltpu

_MAX_LEN = 8192
_PE_DIM = 128
_D_MODEL = 1024
_BATCH = 4
_OUT_D = _D_MODEL + _PE_DIM
_S = 2048
_N = _BATCH * _MAX_LEN // _S
_H = _S // 2


def _body(x_hbm, pe_hbm, out_hbm, bufx, bufp, bufo, sem_in, sem_o1, sem_o2):
    i = pl.program_id(0)

    def in_descs(j):
        slot = j % 2
        t0 = (j % (_MAX_LEN // _S)) * _S
        cx = pltpu.make_async_copy(
            x_hbm.at[pl.ds(j * _S, _S), :], bufx.at[slot], sem_in.at[slot])
        cp = pltpu.make_async_copy(
            pe_hbm.at[pl.ds(t0, _S), :], bufp.at[slot], sem_in.at[slot])
        return cx, cp

    def out_descs(j):
        slot = j % 3
        c1 = pltpu.make_async_copy(
            bufo.at[slot, pl.ds(0, _H)],
            out_hbm.at[pl.ds(j * _S, _H), :], sem_o1.at[slot])
        c2 = pltpu.make_async_copy(
            bufo.at[slot, pl.ds(_H, _H)],
            out_hbm.at[pl.ds(j * _S + _H, _H), :], sem_o2.at[slot])
        return c1, c2

    @pl.when(i == 0)
    def _():
        for c in in_descs(0):
            c.start()
        for c in in_descs(1):
            c.start()

    @pl.when(i >= 3)
    def _():
        for c in out_descs(i - 3):
            c.wait()

    for c in in_descs(i):
        c.wait()

    oslot = i % 3
    islot = i % 2
    bufo[oslot, :, :_D_MODEL] = bufx[islot]
    bufo[oslot, :, _D_MODEL:] = bufp[islot]

    for c in out_descs(i):
        c.start()

    @pl.when(i + 2 < _N)
    def _():
        for c in in_descs(i + 2):
            c.start()

    @pl.when(i == _N - 1)
    def _():
        for j in range(_N - 3, _N):
            for c in out_descs(j):
                c.wait()


def kernel(x, pe_table):
    batch, max_len, d_model = x.shape
    x2 = x.reshape(batch * max_len, d_model)
    out = pl.pallas_call(
        _body,
        grid=(_N,),
        in_specs=[
            pl.BlockSpec(memory_space=pl.ANY),
            pl.BlockSpec(memory_space=pl.ANY),
        ],
        out_specs=pl.BlockSpec(memory_space=pl.ANY),
        out_shape=jax.ShapeDtypeStruct((batch * max_len, _OUT_D), jnp.float32),
        scratch_shapes=[
            pltpu.VMEM((2, _S, _D_MODEL), jnp.float32),
            pltpu.VMEM((2, _S, _PE_DIM), jnp.float32),
            pltpu.VMEM((3, _S, _OUT_D), jnp.float32),
            pltpu.SemaphoreType.DMA((2,)),
            pltpu.SemaphoreType.DMA((3,)),
            pltpu.SemaphoreType.DMA((3,)),
        ],
    )(x2, pe_table)
    return out.reshape(batch, max_len, _OUT_D)


# speedup vs baseline: 1.0442x; 1.0442x over previous
import jax
import jax.numpy as jnp
from jax.experimental import pallas as pl

_D_MODEL = 1024
_SEQ_BLK = 2048


def _body(x_ref, pe_ref, o_ref):
    o_ref[:, :, :_D_MODEL] = x_ref[...]
    o_ref[:, :, _D_MODEL:] = pe_ref[...][None]


def kernel(x, pe_table):
    batch, max_len, d_model = x.shape
    pe_dim = pe_table.shape[1]
    grid = (max_len // _SEQ_BLK, batch)
    return pl.pallas_call(
        _body,
        grid=grid,
        in_specs=[
            pl.BlockSpec((1, _SEQ_BLK, d_model), lambda s, b: (b, s, 0)),
            pl.BlockSpec((_SEQ_BLK, pe_dim), lambda s, b: (s, 0)),
        ],
        out_specs=pl.BlockSpec((1, _SEQ_BLK, d_model + pe_dim),
                               lambda s, b: (b, s, 0)),
        out_shape=jax.ShapeDtypeStruct((batch, max_len, d_model + pe_dim),
                                       x.dtype),
    )(x, pe_table)
